# baseline (speedup 1.0000x reference)
import jax
import jax.numpy as jnp
from jax import lax
from jax.experimental import pallas as pl
from jax.experimental.pallas import tpu as pltpu
from jax.experimental.pallas import tpu_sc as plsc

_N = 131072
_K = 256
_NBINS = 32

_NC = 2
_NS = 16
_NW = _NC * _NS
_ROWS_W = _N // _NW
_CH = 128
_NCHUNK = _ROWS_W // _CH
_CNT_PAD = (_NBINS + 1) * _K

_MM_ROWS = 16384


def _minmax_body(x_ref, sv_ref, ov_ref, hv_ref, amn_ref, amx_ref):
    i = pl.program_id(0)
    x = x_ref[...].reshape(_MM_ROWS // 8, 8, _K)
    bmn = jnp.min(x, axis=0)
    bmx = jnp.max(x, axis=0)

    @pl.when(i == 0)
    def _():
        amn_ref[...] = bmn
        amx_ref[...] = bmx

    @pl.when(i > 0)
    def _():
        amn_ref[...] = jnp.minimum(amn_ref[...], bmn)
        amx_ref[...] = jnp.maximum(amx_ref[...], bmx)

    @pl.when(i == _N // _MM_ROWS - 1)
    def _():
        r0 = jnp.min(amn_ref[...])
        r1 = jnp.max(amx_ref[...])
        denom = r1 - r0
        ok = denom > 0.0
        scale = jnp.where(ok, jnp.float32(_NBINS) / denom,
                          jnp.float32(0.0)) * jnp.float32(_K)
        off = jnp.where(ok, -r0 * scale, jnp.float32(_NBINS * _K))
        sv_ref[...] = jnp.full((1, 128), scale, jnp.float32)
        ov_ref[...] = jnp.full((1, 128), off, jnp.float32)
        hv_ref[...] = jnp.full((1, 128), r1, jnp.float32)


def _minmax(x):
    vec = jax.ShapeDtypeStruct((1, 128), jnp.float32)
    return pl.pallas_call(
        _minmax_body,
        grid=(_N // _MM_ROWS,),
        in_specs=[pl.BlockSpec((_MM_ROWS, _K), lambda i: (i, 0))],
        out_specs=[
            pl.BlockSpec((1, 128), lambda i: (0, 0)),
            pl.BlockSpec((1, 128), lambda i: (0, 0)),
            pl.BlockSpec((1, 128), lambda i: (0, 0)),
        ],
        out_shape=[vec, vec, vec],
        scratch_shapes=[
            pltpu.VMEM((8, _K), jnp.float32),
            pltpu.VMEM((8, _K), jnp.float32),
        ],
    )(x)


def _hist_body(x_hbm, sv_hbm, ov_hbm, hv_hbm, out_hbm, buf, counts,
               sv_v, ov_v, hv_v, sem0, sem1):
    cid = lax.axis_index("c")
    sid = lax.axis_index("s")
    wid = sid * _NC + cid
    base = wid * _ROWS_W
    sems = (sem0, sem1)

    def issue(ci, s):
        pltpu.async_copy(x_hbm.at[pl.ds(base + ci * _CH, _CH)], buf.at[s], sems[s])

    issue(0, 0)
    issue(1, 1)

    pltpu.sync_copy(sv_hbm, sv_v)
    pltpu.sync_copy(ov_hbm, ov_v)
    pltpu.sync_copy(hv_hbm, hv_v)
    scale = sv_v[...]
    off = ov_v[...]
    hi = hv_v[...]

    zeros16 = jnp.zeros((16,), jnp.float32)

    @plsc.parallel_loop(0, _CNT_PAD // 16, unroll=8)
    def _zero(i):
        counts[pl.ds(i * 16, 16)] = zeros16

    ones16 = jnp.ones((16,), jnp.float32)
    iota16 = lax.iota(jnp.int32, 16)

    def pair_body(p, _):
        for s in range(2):
            ci = p * 2 + s
            pltpu.make_async_copy(
                x_hbm.at[pl.ds(0, _CH)], buf.at[s], sems[s]).wait()

            for h in range(2):
                @plsc.parallel_loop(0, _CH * (_K // 32), unroll=8)
                def _vreg(v, h=h):
                    r = v // 8
                    c = (v & 7) * 16 + h * (_K // 2)
                    x = buf[s, r, pl.ds(c, 16)]
                    t = x * scale + off
                    bi256 = t.astype(jnp.int32) & jnp.int32(~255)
                    flat = bi256 + (c + iota16)
                    plsc.addupdate_scatter(counts, [flat], ones16,
                                           mask=x < hi)

            @pl.when(ci + 2 < _NCHUNK)
            def _():
                issue(ci + 2, s)

        return 0

    lax.fori_loop(0, _NCHUNK // 2, pair_body, 0)

    pltpu.sync_copy(counts.at[pl.ds(0, _NBINS * _K)], out_hbm.at[wid])


def _hist(x, scale_v, off_v, hi_v):
    mesh = plsc.VectorSubcoreMesh(core_axis_name="c", subcore_axis_name="s")
    f = pl.kernel(
        _hist_body,
        mesh=mesh,
        out_type=jax.ShapeDtypeStruct((_NW, _NBINS * _K), jnp.float32),
        scratch_types=[
            pltpu.VMEM((2, _CH, _K), jnp.float32),
            pltpu.VMEM((_CNT_PAD,), jnp.float32),
            pltpu.VMEM((16,), jnp.float32),
            pltpu.VMEM((16,), jnp.float32),
            pltpu.VMEM((16,), jnp.float32),
            pltpu.SemaphoreType.DMA,
            pltpu.SemaphoreType.DMA,
        ],
        compiler_params=pltpu.CompilerParams(needs_layout_passes=False),
    )
    return f(x, scale_v, off_v, hi_v)


def _entropy_body(parts_ref, pi_ref, out_ref):
    c = jnp.sum(parts_ref[...], axis=0)
    d = c * (1.0 / _N)
    term = -d * jnp.log(d + 1e-07)
    ent = jnp.sum(term, axis=0)
    w = 1.0 - pi_ref[...]
    out_ref[0, 0] = jnp.sum(w * ent[None, :]) / 0.6931471805599453


def _entropy(parts, pi):
    return pl.pallas_call(
        _entropy_body,
        out_specs=pl.BlockSpec(memory_space=pltpu.SMEM),
        out_shape=jax.ShapeDtypeStruct((1, 1), jnp.float32),
    )(parts, pi)


def kernel(mean_t, pi):
    sv, ov, hv = _minmax(mean_t)
    scale_v = sv[0, :16]
    off_v = ov[0, :16]
    hi_v = hv[0, :16]
    parts = _hist(mean_t, scale_v, off_v, hi_v)
    parts3 = parts.reshape(_NW, _NBINS, _K)
    ixt = _entropy(parts3, pi)
    return ixt.reshape(1)


# speedup vs baseline: 1.0087x; 1.0087x over previous
import jax
import jax.numpy as jnp
from jax import lax
from jax.experimental import pallas as pl
from jax.experimental.pallas import tpu as pltpu
from jax.experimental.pallas import tpu_sc as plsc

_N = 131072
_K = 256
_NBINS = 32

_NC = 2
_NS = 16
_NW = _NC * _NS
_ROWS_W = _N // _NW
_CH = 128
_NCHUNK = _ROWS_W // _CH
_CNT_PAD = (_NBINS + 1) * _K

_MM_ROWS = 8192


def _minmax_body(x_ref, sv_ref, ov_ref, hv_ref, amn_ref, amx_ref):
    i = pl.program_id(0)
    x = x_ref[...].reshape(_MM_ROWS // 8, 8, _K)
    bmn = jnp.min(x, axis=0)
    bmx = jnp.max(x, axis=0)

    @pl.when(i == 0)
    def _():
        amn_ref[...] = bmn
        amx_ref[...] = bmx

    @pl.when(i > 0)
    def _():
        amn_ref[...] = jnp.minimum(amn_ref[...], bmn)
        amx_ref[...] = jnp.maximum(amx_ref[...], bmx)

    @pl.when(i == _N // _MM_ROWS - 1)
    def _():
        r0 = jnp.min(amn_ref[...])
        r1 = jnp.max(amx_ref[...])
        denom = r1 - r0
        ok = denom > 0.0
        scale = jnp.where(ok, jnp.float32(_NBINS) / denom,
                          jnp.float32(0.0)) * jnp.float32(_K)
        off = jnp.where(ok, -r0 * scale, jnp.float32(_NBINS * _K))
        sv_ref[...] = jnp.full((1, 128), scale, jnp.float32)
        ov_ref[...] = jnp.full((1, 128), off, jnp.float32)
        hv_ref[...] = jnp.full((1, 128), r1, jnp.float32)


def _minmax(x):
    vec = jax.ShapeDtypeStruct((1, 128), jnp.float32)
    return pl.pallas_call(
        _minmax_body,
        grid=(_N // _MM_ROWS,),
        in_specs=[pl.BlockSpec((_MM_ROWS, _K), lambda i: (i, 0))],
        out_specs=[
            pl.BlockSpec((1, 128), lambda i: (0, 0)),
            pl.BlockSpec((1, 128), lambda i: (0, 0)),
            pl.BlockSpec((1, 128), lambda i: (0, 0)),
        ],
        out_shape=[vec, vec, vec],
        scratch_shapes=[
            pltpu.VMEM((8, _K), jnp.float32),
            pltpu.VMEM((8, _K), jnp.float32),
        ],
    )(x)


def _hist_body(x_hbm, sv_hbm, ov_hbm, hv_hbm, out_hbm, buf, counts,
               sv_v, ov_v, hv_v, sem0, sem1):
    cid = lax.axis_index("c")
    sid = lax.axis_index("s")
    wid = sid * _NC + cid
    base = wid * _ROWS_W
    sems = (sem0, sem1)

    def issue(ci, s):
        pltpu.async_copy(x_hbm.at[pl.ds(base + ci * _CH, _CH)], buf.at[s], sems[s])

    issue(0, 0)
    issue(1, 1)

    pltpu.sync_copy(sv_hbm, sv_v)
    pltpu.sync_copy(ov_hbm, ov_v)
    pltpu.sync_copy(hv_hbm, hv_v)
    scale = sv_v[...]
    off = ov_v[...]
    hi = hv_v[...]

    zeros16 = jnp.zeros((16,), jnp.float32)

    @plsc.parallel_loop(0, _CNT_PAD // 16, unroll=8)
    def _zero(i):
        counts[pl.ds(i * 16, 16)] = zeros16

    ones16 = jnp.ones((16,), jnp.float32)
    iota16 = lax.iota(jnp.int32, 16)

    def pair_body(p, _):
        for s in range(2):
            ci = p * 2 + s
            pltpu.make_async_copy(
                x_hbm.at[pl.ds(0, _CH)], buf.at[s], sems[s]).wait()

            for h in range(2):
                @plsc.parallel_loop(0, _CH * (_K // 32), unroll=8)
                def _vreg(v, h=h):
                    r = v // 8
                    c = (v & 7) * 16 + h * (_K // 2)
                    x = buf[s, r, pl.ds(c, 16)]
                    t = x * scale + off
                    bi256 = t.astype(jnp.int32) & jnp.int32(~255)
                    flat = bi256 + (c + iota16)
                    plsc.addupdate_scatter(counts, [flat], ones16,
                                           mask=x < hi)

            @pl.when(ci + 2 < _NCHUNK)
            def _():
                issue(ci + 2, s)

        return 0

    lax.fori_loop(0, _NCHUNK // 2, pair_body, 0)

    pltpu.sync_copy(counts.at[pl.ds(0, _NBINS * _K)], out_hbm.at[wid])


def _hist(x, scale_v, off_v, hi_v):
    mesh = plsc.VectorSubcoreMesh(core_axis_name="c", subcore_axis_name="s")
    f = pl.kernel(
        _hist_body,
        mesh=mesh,
        out_type=jax.ShapeDtypeStruct((_NW, _NBINS * _K), jnp.float32),
        scratch_types=[
            pltpu.VMEM((2, _CH, _K), jnp.float32),
            pltpu.VMEM((_CNT_PAD,), jnp.float32),
            pltpu.VMEM((16,), jnp.float32),
            pltpu.VMEM((16,), jnp.float32),
            pltpu.VMEM((16,), jnp.float32),
            pltpu.SemaphoreType.DMA,
            pltpu.SemaphoreType.DMA,
        ],
        compiler_params=pltpu.CompilerParams(needs_layout_passes=False),
    )
    return f(x, scale_v, off_v, hi_v)


def _entropy_body(parts_ref, pi_ref, out_ref):
    c = jnp.sum(parts_ref[...], axis=0)
    d = c * (1.0 / _N)
    term = -d * jnp.log(d + 1e-07)
    ent = jnp.sum(term, axis=0)
    w = 1.0 - pi_ref[...]
    out_ref[0, 0] = jnp.sum(w * ent[None, :]) / 0.6931471805599453


def _entropy(parts, pi):
    return pl.pallas_call(
        _entropy_body,
        out_specs=pl.BlockSpec(memory_space=pltpu.SMEM),
        out_shape=jax.ShapeDtypeStruct((1, 1), jnp.float32),
    )(parts, pi)


def kernel(mean_t, pi):
    sv, ov, hv = _minmax(mean_t)
    scale_v = sv[0, :16]
    off_v = ov[0, :16]
    hi_v = hv[0, :16]
    parts = _hist(mean_t, scale_v, off_v, hi_v)
    parts3 = parts.reshape(_NW, _NBINS, _K)
    ixt = _entropy(parts3, pi)
    return ixt.reshape(1)
